# baseline (speedup 1.0000x reference)
import functools

import jax
import jax.numpy as jnp
from jax import lax
from jax.experimental import pallas as pl
from jax.experimental.pallas import tpu as pltpu
from jax.experimental.pallas import tpu_sc as plsc

_NC = 2
_NS = 16
_NW = _NC * _NS
_L = 16
_NBUF = 6


def _sc_gather_add_t(x4, lab3, table):
    t_dim, dt, nb, tile = x4.shape
    v, d = table.shape
    bw = 128
    assert nb == _NW and tile == 1024 and d == 64

    mesh = plsc.VectorSubcoreMesh(
        core_axis_name="c", subcore_axis_name="s",
        num_cores=_NC, num_subcores=_NS)

    @functools.partial(
        pl.kernel,
        out_type=jax.ShapeDtypeStruct((t_dim, dt, nb, tile), jnp.float32),
        mesh=mesh,
        compiler_params=pltpu.CompilerParams(
            use_tc_tiling_on_sc=False, needs_layout_passes=False),
        scratch_types=(
            [pltpu.VMEM((t_dim, bw), jnp.int32)]
            + [pltpu.VMEM((bw, d), jnp.float32) for _ in range(_NBUF)]
            + [pltpu.VMEM((dt, tile), jnp.float32) for _ in range(_NBUF)]
            + [pltpu.SemaphoreType.DMA for _ in range(3 * _NBUF)]
        ),
    )
    def k(x_hbm, lab_hbm, tab_hbm, out_hbm, *scr):
        idx_all = scr[0]
        g_v = scr[1:1 + _NBUF]
        xo_v = scr[1 + _NBUF:1 + 2 * _NBUF]
        s_x = scr[1 + 2 * _NBUF:1 + 3 * _NBUF]
        s_g = scr[1 + 3 * _NBUF:1 + 4 * _NBUF]
        s_o = scr[1 + 4 * _NBUF:1 + 5 * _NBUF]
        wid = lax.axis_index("s") * _NC + lax.axis_index("c")
        lane = lax.iota(jnp.int32, 16)

        def load(t, p):
            pltpu.async_copy(x_hbm.at[t, :, wid], xo_v[p], s_x[p])

        def wait_x(p):
            pltpu.make_async_copy(x_hbm.at[0, :, wid], xo_v[p], s_x[p]).wait()

        def gather(t, p):
            pltpu.async_copy(tab_hbm.at[idx_all.at[t]], g_v[p], s_g[p])

        def wait_gather(p):
            pltpu.make_async_copy(
                tab_hbm.at[idx_all.at[0]], g_v[p], s_g[p]).wait()

        def store(t, p):
            pltpu.async_copy(xo_v[p], out_hbm.at[t, :, wid], s_o[p])

        def wait_store(p):
            pltpu.make_async_copy(xo_v[p], out_hbm.at[0, :, wid], s_o[p]).wait()

        def compute(p):
            g = g_v[p]
            xo = xo_v[p]
            d2n = d // dt
            npair = 2

            @plsc.parallel_loop(0, (bw // _L // npair) * d, 1, unroll=8)
            def body(i, g=g, xo=xo):
                bg0 = (i // d) * npair
                dd = i % d
                d1 = dd // d2n
                d2 = dd % d2n
                ddvec = jnp.full((16,), dd, jnp.int32)
                for j in range(npair):
                    bg = bg0 + j
                    vals = plsc.load_gather(g, [bg * _L + lane, ddvec])
                    plsc.addupdate(
                        xo.at[d1, pl.ds(d2 * 128 + bg * _L, _L)], vals)

        def step(t, p, drain_store, prefetch, fire_gather):
            q3 = (p + 3) % _NBUF
            if drain_store:
                wait_store(q3)
            if prefetch:
                load(t + 3, q3)
            if fire_gather:
                gather(t + 2, (p + 2) % _NBUF)
            wait_gather(p)
            wait_x(p)
            compute(p)
            store(t, p)

        nsteps = t_dim
        rounds = nsteps // _NBUF
        pltpu.async_copy(lab_hbm.at[:, wid, :], idx_all, s_x[0])
        pltpu.make_async_copy(lab_hbm.at[:, wid, :], idx_all, s_x[0]).wait()
        load(0, 0)
        load(1, 1)
        load(2, 2)
        gather(0, 0)
        gather(1, 1)

        for p in range(_NBUF):
            step(p, p, drain_store=(p >= 3), prefetch=True, fire_gather=True)

        def round_body(gr, carry):
            t0 = gr * _NBUF
            for p in range(_NBUF):
                step(t0 + p, p, drain_store=True, prefetch=True,
                     fire_gather=True)
            return carry

        lax.fori_loop(1, rounds - 1, round_body, 0)

        t0 = (rounds - 1) * _NBUF
        tail = nsteps - t0
        for i in range(tail):
            t = t0 + i
            step(t, (t0 + i) % _NBUF,
                 drain_store=(t + 3 < nsteps),
                 prefetch=(t + 3 < nsteps),
                 fire_gather=(t + 2 < nsteps))
        for i in range(tail - _NBUF, tail):
            wait_store((t0 + i) % _NBUF)

    return k(x4, lab3, table)


def kernel(x, cluster_labels, table):
    b, lp1, d = x.shape
    x4 = (x.reshape(32, 128, lp1, 8, 8)
          .transpose(2, 3, 0, 4, 1)
          .reshape(lp1, 8, 32, 1024))
    labels = jnp.concatenate(
        [jnp.zeros((1, b), dtype=cluster_labels.dtype),
         cluster_labels.T], axis=0)
    lab3 = labels.reshape(lp1, 32, 128)
    out4 = _sc_gather_add_t(x4, lab3, table)
    return (out4.reshape(lp1, 8, 32, 8, 128)
            .transpose(2, 4, 0, 1, 3)
            .reshape(b, lp1, d))


# speedup vs baseline: 2.3523x; 2.3523x over previous
import functools

import jax
import jax.numpy as jnp
from jax import lax
from jax.experimental import pallas as pl
from jax.experimental.pallas import tpu as pltpu
from jax.experimental.pallas import tpu_sc as plsc

_NC = 2
_NS = 16
_NW = _NC * _NS
_L = 16
_NBUF = 6


def _sc_gather_add_t(x4, lab3, table):
    t_dim, dt, nb, tile = x4.shape
    v, d = table.shape
    bw = 128
    assert nb == _NW and tile == 1024 and d == 64

    mesh = plsc.VectorSubcoreMesh(
        core_axis_name="c", subcore_axis_name="s",
        num_cores=_NC, num_subcores=_NS)

    @functools.partial(
        pl.kernel,
        out_type=jax.ShapeDtypeStruct((t_dim, dt, nb, tile), jnp.float32),
        mesh=mesh,
        compiler_params=pltpu.CompilerParams(
            use_tc_tiling_on_sc=False, needs_layout_passes=False),
        scratch_types=(
            [pltpu.VMEM((t_dim, bw), jnp.int32)]
            + [pltpu.VMEM((bw, d), jnp.float32) for _ in range(_NBUF)]
            + [pltpu.VMEM((dt, tile), jnp.float32) for _ in range(_NBUF)]
            + [pltpu.SemaphoreType.DMA for _ in range(3 * _NBUF)]
        ),
    )
    def k(x_hbm, lab_hbm, tab_hbm, out_hbm, *scr):
        idx_all = scr[0]
        g_v = scr[1:1 + _NBUF]
        xo_v = scr[1 + _NBUF:1 + 2 * _NBUF]
        s_x = scr[1 + 2 * _NBUF:1 + 3 * _NBUF]
        s_g = scr[1 + 3 * _NBUF:1 + 4 * _NBUF]
        s_o = scr[1 + 4 * _NBUF:1 + 5 * _NBUF]
        wid = lax.axis_index("s") * _NC + lax.axis_index("c")
        lane = lax.iota(jnp.int32, 16)

        def load(t, p):
            pltpu.async_copy(x_hbm.at[t, :, wid], xo_v[p], s_x[p])

        def wait_x(p):
            pltpu.make_async_copy(x_hbm.at[0, :, wid], xo_v[p], s_x[p]).wait()

        def gather(t, p):
            pltpu.async_copy(tab_hbm.at[idx_all.at[t]], g_v[p], s_g[p])

        def wait_gather(p):
            pltpu.make_async_copy(
                tab_hbm.at[idx_all.at[0]], g_v[p], s_g[p]).wait()

        def store(t, p):
            pltpu.async_copy(xo_v[p], out_hbm.at[t, :, wid], s_o[p])

        def wait_store(p):
            pltpu.make_async_copy(xo_v[p], out_hbm.at[0, :, wid], s_o[p]).wait()

        def compute(p):
            g = g_v[p]
            xo = xo_v[p]
            ntile = (bw // _L) * (d // _L)

            @plsc.parallel_loop(0, _L * ntile, 1, unroll=8)
            def body(i, g=g, xo=xo):
                k = i // ntile
                ti = i % ntile
                b0 = (ti // (d // _L)) * _L
                dd0 = (ti % (d // _L)) * _L
                rowv = b0 + lane
                ddv = dd0 + ((lane + k) & (_L - 1))
                vals = plsc.load_gather(g, [rowv, ddv])
                d1v = ddv // 8
                colv = (ddv % 8) * 128 + rowv
                plsc.addupdate_scatter(xo, [d1v, colv], vals)

        def step(t, p, drain_store, prefetch, fire_gather):
            q3 = (p + 3) % _NBUF
            if drain_store:
                wait_store(q3)
            if prefetch:
                load(t + 3, q3)
            if fire_gather:
                gather(t + 2, (p + 2) % _NBUF)
            wait_gather(p)
            wait_x(p)
            compute(p)
            store(t, p)

        nsteps = t_dim
        rounds = nsteps // _NBUF
        pltpu.async_copy(lab_hbm.at[:, wid, :], idx_all, s_x[0])
        pltpu.make_async_copy(lab_hbm.at[:, wid, :], idx_all, s_x[0]).wait()
        load(0, 0)
        load(1, 1)
        load(2, 2)
        gather(0, 0)
        gather(1, 1)

        for p in range(_NBUF):
            step(p, p, drain_store=(p >= 3), prefetch=True, fire_gather=True)

        def round_body(gr, carry):
            t0 = gr * _NBUF
            for p in range(_NBUF):
                step(t0 + p, p, drain_store=True, prefetch=True,
                     fire_gather=True)
            return carry

        lax.fori_loop(1, rounds - 1, round_body, 0)

        t0 = (rounds - 1) * _NBUF
        tail = nsteps - t0
        for i in range(tail):
            t = t0 + i
            step(t, (t0 + i) % _NBUF,
                 drain_store=(t + 3 < nsteps),
                 prefetch=(t + 3 < nsteps),
                 fire_gather=(t + 2 < nsteps))
        for i in range(tail - _NBUF, tail):
            wait_store((t0 + i) % _NBUF)

    return k(x4, lab3, table)


def kernel(x, cluster_labels, table):
    b, lp1, d = x.shape
    x4 = (x.reshape(32, 128, lp1, 8, 8)
          .transpose(2, 3, 0, 4, 1)
          .reshape(lp1, 8, 32, 1024))
    labels = jnp.concatenate(
        [jnp.zeros((1, b), dtype=cluster_labels.dtype),
         cluster_labels.T], axis=0)
    lab3 = labels.reshape(lp1, 32, 128)
    out4 = _sc_gather_add_t(x4, lab3, table)
    return (out4.reshape(lp1, 8, 32, 8, 128)
            .transpose(2, 4, 0, 1, 3)
            .reshape(b, lp1, d))
